# Initial kernel scaffold; baseline (speedup 1.0000x reference)
import jax
import jax.numpy as jnp
from jax.experimental import pallas as pl


def kernel(vertices, faces, attributes):
    raise NotImplementedError("write your pallas kernel here")


# speedup vs baseline: 10.5079x; 10.5079x over previous
import functools

import jax
import jax.numpy as jnp
from jax import lax
from jax.experimental import pallas as pl
from jax.experimental.pallas import tpu as pltpu
from jax.experimental.pallas import tpu_sc as plsc

H = 512
W = 512
BZ = 4
NV = 5023
NF = 9976
D = 8

NW = 32
NFT = BZ * NF
FPW = 1248
NFP = NW * FPW
NVT = BZ * NV
NVP = 20096
PIX = BZ * H * W
PPW = PIX // NW
CH2 = 2496
CH3 = 2048
NCH = 9
ROWW = 16

_mesh = plsc.VectorSubcoreMesh(core_axis_name="c", subcore_axis_name="s")
_cparams = pltpu.CompilerParams(needs_layout_passes=False,
                                use_tc_tiling_on_sc=False)


def _wid():
    return lax.axis_index("s") * 2 + lax.axis_index("c")


@functools.partial(
    pl.kernel,
    mesh=_mesh,
    compiler_params=_cparams,
    out_type=(
        jax.ShapeDtypeStruct((NFP,), jnp.int32),
        jax.ShapeDtypeStruct((NFP,), jnp.float32),
        jax.ShapeDtypeStruct((NFP,), jnp.int32),
        jax.ShapeDtypeStruct((NFP * ROWW,), jnp.float32),
    ),
    scratch_types=[
        pltpu.VMEM((FPW,), jnp.int32),
        pltpu.VMEM((FPW,), jnp.int32),
        pltpu.VMEM((FPW,), jnp.int32),
        pltpu.VMEM((NVP,), jnp.float32),
        pltpu.VMEM((NVP,), jnp.float32),
        pltpu.VMEM((NVP,), jnp.float32),
        pltpu.VMEM((24 * FPW,), jnp.float32),
        pltpu.VMEM((FPW * ROWW,), jnp.float32),
        pltpu.VMEM((FPW,), jnp.int32),
        pltpu.VMEM((FPW,), jnp.float32),
        pltpu.VMEM((FPW,), jnp.int32),
    ],
)
def _phase1(i0_hbm, i1_hbm, i2_hbm, vx_hbm, vy_hbm, vz_hbm, a_hbm,
            lin_out, d_out, fid_out, rows_out,
            i0_v, i1_v, i2_v, vx_v, vy_v, vz_v, a_v, rowbuf,
            lin_v, d_v, fid_v):
    wid = _wid()
    base = wid * FPW
    pltpu.sync_copy(i0_hbm.at[pl.ds(base, FPW)], i0_v)
    pltpu.sync_copy(i1_hbm.at[pl.ds(base, FPW)], i1_v)
    pltpu.sync_copy(i2_hbm.at[pl.ds(base, FPW)], i2_v)
    pltpu.sync_copy(vx_hbm, vx_v)
    pltpu.sync_copy(vy_hbm, vy_v)
    pltpu.sync_copy(vz_hbm, vz_v)
    for r in range(24):
        pltpu.sync_copy(a_hbm.at[pl.ds(r * NFP + base, FPW)],
                        a_v.at[pl.ds(r * FPW, FPW)])

    zero16 = jnp.zeros((16,), jnp.float32)

    def zrow(j, carry):
        rowbuf[pl.ds(j * 16, 16)] = zero16
        return carry

    lax.fori_loop(0, FPW * ROWW // 16, zrow, 0)

    third = jnp.float32(1.0 / 3.0)
    ones16 = jnp.full((16,), 1.0, jnp.float32)

    def body(i, carry):
        o = i * 16
        lanes = lax.iota(jnp.int32, 16)
        gf = base + o + lanes
        validf = gf < NFT
        g0 = i0_v[pl.ds(o, 16)]
        g1 = i1_v[pl.ds(o, 16)]
        g2 = i2_v[pl.ds(o, 16)]
        x0 = plsc.load_gather(vx_v, [g0]) * (W / 2) + (W / 2)
        x1 = plsc.load_gather(vx_v, [g1]) * (W / 2) + (W / 2)
        x2 = plsc.load_gather(vx_v, [g2]) * (W / 2) + (W / 2)
        y0 = plsc.load_gather(vy_v, [g0]) * (H / 2) + (H / 2)
        y1 = plsc.load_gather(vy_v, [g1]) * (H / 2) + (H / 2)
        y2 = plsc.load_gather(vy_v, [g2]) * (H / 2) + (H / 2)
        z0 = plsc.load_gather(vz_v, [g0]) * (W / 2)
        z1 = plsc.load_gather(vz_v, [g1]) * (W / 2)
        z2 = plsc.load_gather(vz_v, [g2]) * (W / 2)
        cx = ((x0 + x1) + x2) / 3.0
        cy = ((y0 + y1) + y2) / 3.0
        cz = ((z0 + z1) + z2) / 3.0
        px = jnp.clip(cx.astype(jnp.int32), 0, W - 1)
        py = jnp.clip(cy.astype(jnp.int32), 0, H - 1)
        b = ((gf >= NF).astype(jnp.int32)
             + (gf >= 2 * NF).astype(jnp.int32)
             + (gf >= 3 * NF).astype(jnp.int32))
        lin = (b * H + py) * W + px
        lin_v[pl.ds(o, 16)] = lin
        d_v[pl.ds(o, 16)] = jnp.where(validf, cz, jnp.float32(1e30))
        fid_v[pl.ds(o, 16)] = jnp.where(validf, gf, -1)
        rowb = (o + lanes) * ROWW
        for c in range(8):
            a0 = a_v[pl.ds((0 * 8 + c) * FPW + o, 16)]
            a1 = a_v[pl.ds((1 * 8 + c) * FPW + o, 16)]
            a2 = a_v[pl.ds((2 * 8 + c) * FPW + o, 16)]
            m = (a0 * third + a1 * third) + a2 * third
            plsc.store_scatter(rowbuf, [rowb + c], m)
        plsc.store_scatter(rowbuf, [rowb + 8], ones16)
        return carry

    lax.fori_loop(0, FPW // 16, body, 0)

    pltpu.sync_copy(lin_v, lin_out.at[pl.ds(base, FPW)])
    pltpu.sync_copy(d_v, d_out.at[pl.ds(base, FPW)])
    pltpu.sync_copy(fid_v, fid_out.at[pl.ds(base, FPW)])
    pltpu.sync_copy(rowbuf, rows_out.at[pl.ds(base * ROWW, FPW * ROWW)])


@functools.partial(
    pl.kernel,
    mesh=_mesh,
    compiler_params=_cparams,
    out_type=jax.ShapeDtypeStruct((PIX,), jnp.int32),
    scratch_types=[
        pltpu.VMEM((PPW,), jnp.float32),
        pltpu.VMEM((PPW,), jnp.int32),
        pltpu.VMEM((CH2,), jnp.int32),
        pltpu.VMEM((CH2,), jnp.float32),
        pltpu.VMEM((CH2,), jnp.int32),
    ],
)
def _phase2(lin_hbm, d_hbm, fid_hbm, fidbuf_out,
            dep_loc, fid_loc, cl, cd, cf):
    wid = _wid()
    lo = wid * PPW
    init_d = jnp.full((16,), 1e6, jnp.float32)
    init_f = jnp.full((16,), -1, jnp.int32)

    def initb(j, carry):
        dep_loc[pl.ds(j * 16, 16)] = init_d
        fid_loc[pl.ds(j * 16, 16)] = init_f
        return carry

    lax.fori_loop(0, PPW // 16, initb, 0)

    def chunk(cidx, carry):
        co = cidx * CH2
        pltpu.sync_copy(lin_hbm.at[pl.ds(co, CH2)], cl)
        pltpu.sync_copy(d_hbm.at[pl.ds(co, CH2)], cd)
        pltpu.sync_copy(fid_hbm.at[pl.ds(co, CH2)], cf)

        def vb(j, c2):
            lin = cl[pl.ds(j * 16, 16)]
            d = cd[pl.ds(j * 16, 16)]
            fid = cf[pl.ds(j * 16, 16)]
            mine = (lin >= lo) & (lin < lo + PPW)
            loc = jnp.where(mine, lin - lo, 0)
            gd = plsc.load_gather(dep_loc, [loc])
            gfi = plsc.load_gather(fid_loc, [loc])
            m = mine & ((d < gd) | ((d == gd) & (fid > gfi)))

            def wcond(mm):
                return jnp.any(mm)

            def wbody(mm):
                plsc.store_scatter(dep_loc, [loc], d, mask=mm)
                plsc.store_scatter(fid_loc, [loc], fid, mask=mm)
                gd2 = plsc.load_gather(dep_loc, [loc])
                gf2 = plsc.load_gather(fid_loc, [loc])
                return mine & ((d < gd2) | ((d == gd2) & (fid > gf2)))

            lax.while_loop(wcond, wbody, m)
            return c2

        lax.fori_loop(0, CH2 // 16, vb, 0)
        return carry

    lax.fori_loop(0, NFP // CH2, chunk, 0)
    pltpu.sync_copy(fid_loc, fidbuf_out.at[pl.ds(lo, PPW)])


@functools.partial(
    pl.kernel,
    mesh=_mesh,
    compiler_params=_cparams,
    out_type=jax.ShapeDtypeStruct((BZ * NCH * H * W,), jnp.float32),
    scratch_types=[
        pltpu.VMEM((PPW,), jnp.int32),
        pltpu.VMEM((CH3,), jnp.int32),
        pltpu.VMEM((CH3, ROWW), jnp.float32),
        pltpu.VMEM((CH3,), jnp.float32),
        pltpu.SemaphoreType.DMA,
    ],
)
def _phase3(fidbuf_hbm, rows_hbm, out_hbm,
            fid_v, idx_v, rows_v, plane, sem):
    wid = _wid()
    lo = wid * PPW
    b = wid // 8
    blk = wid - b * 8
    obase = b * (NCH * H * W) + blk * PPW
    pltpu.sync_copy(fidbuf_hbm.at[pl.ds(lo, PPW)], fid_v)

    def chunk(t, carry):
        to = t * CH3

        def bi(j, c2):
            f = fid_v[pl.ds(to + j * 16, 16)]
            idx_v[pl.ds(j * 16, 16)] = jnp.where(f > -1, f, 0)
            return c2

        lax.fori_loop(0, CH3 // 16, bi, 0)
        pltpu.async_copy(rows_hbm.at[idx_v], rows_v, sem).wait()

        def ch(c, c2):
            def pv(j, c3):
                lanes = lax.iota(jnp.int32, 16)
                f = fid_v[pl.ds(to + j * 16, 16)]
                vals = plsc.load_gather(
                    rows_v, [j * 16 + lanes, jnp.zeros((16,), jnp.int32) + c])
                plane[pl.ds(j * 16, 16)] = jnp.where(f > -1, vals, 0.0)
                return c3

            lax.fori_loop(0, CH3 // 16, pv, 0)
            pltpu.sync_copy(plane, out_hbm.at[pl.ds(obase + c * (H * W) + to, CH3)])
            return c2

        lax.fori_loop(0, 8, ch, 0)

        def pv8(j, c2):
            f = fid_v[pl.ds(to + j * 16, 16)]
            plane[pl.ds(j * 16, 16)] = jnp.where(f > -1, jnp.float32(1.0), 0.0)
            return c2

        lax.fori_loop(0, CH3 // 16, pv8, 0)
        pltpu.sync_copy(plane, out_hbm.at[pl.ds(obase + 8 * (H * W) + to, CH3)])
        return carry

    lax.fori_loop(0, PPW // CH3, chunk, 0)


def kernel(vertices, faces, attributes):
    v32 = vertices.astype(jnp.float32)
    offs = (jnp.arange(BZ, dtype=jnp.int32) * NV)[:, None, None]
    gfaces = (faces.astype(jnp.int32) + offs)
    gT = jnp.transpose(gfaces, (2, 0, 1)).reshape(3, NFT)
    gTp = jnp.pad(gT, ((0, 0), (0, NFP - NFT)))
    vT = jnp.transpose(v32, (2, 0, 1)).reshape(3, NVT)
    vTp = jnp.pad(vT, ((0, 0), (0, NVP - NVT)))
    aT = jnp.transpose(attributes.astype(jnp.float32), (2, 3, 0, 1)).reshape(24, NFT)
    aTp = jnp.pad(aT, ((0, 0), (0, NFP - NFT))).reshape(24 * NFP)

    lin, d, fid, rows = _phase1(gTp[0], gTp[1], gTp[2],
                                vTp[0], vTp[1], vTp[2], aTp)
    fidbuf = _phase2(lin, d, fid)
    out = _phase3(fidbuf, rows.reshape(NFP, ROWW))
    return out.reshape(BZ, NCH, H, W)
